# baseline (speedup 1.0000x reference)
import jax
import jax.numpy as jnp
from jax import lax
from jax.experimental import pallas as pl
from jax.experimental.pallas import tpu as pltpu

BLANK = 0
DEPTH = 1000
NUM_SLOTS = 10


def _onehot_body(idx_ref, out_ref, scratch, sems):
    i = pl.program_id(0)
    for k in range(NUM_SLOTS):
        @pl.when(i > 0)
        def _wait_prev():
            pltpu.make_async_copy(
                scratch.at[k],
                out_ref.at[(i - 1) * NUM_SLOTS + k],
                sems.at[k],
            ).wait()

        row = idx_ref[k]
        shifted = jnp.where(row == BLANK, -1, row)
        iota = lax.broadcasted_iota(
            jnp.int32, (DEPTH, idx_ref.shape[2]), 0)
        scratch[k] = (shifted == iota).astype(jnp.float32)

        pltpu.make_async_copy(
            scratch.at[k],
            out_ref.at[i * NUM_SLOTS + k],
            sems.at[k],
        ).start()

    @pl.when(i == pl.num_programs(0) - 1)
    def _drain():
        for k in range(NUM_SLOTS):
            pltpu.make_async_copy(
                scratch.at[k],
                out_ref.at[i * NUM_SLOTS + k],
                sems.at[k],
            ).wait()


def kernel(outputs, outputs_length):
    b, t = outputs.shape
    idx3 = outputs.astype(jnp.int32).T.reshape(t, 1, b)
    one_hot_t = pl.pallas_call(
        _onehot_body,
        grid=(t // NUM_SLOTS,),
        in_specs=[pl.BlockSpec((NUM_SLOTS, 1, b), lambda i: (i, 0, 0))],
        out_specs=pl.BlockSpec(memory_space=pl.ANY),
        out_shape=jax.ShapeDtypeStruct((t, DEPTH, b), jnp.float32),
        scratch_shapes=[
            pltpu.VMEM((NUM_SLOTS, DEPTH, b), jnp.float32),
            pltpu.SemaphoreType.DMA((NUM_SLOTS,)),
        ],
    )(idx3)
    return (jnp.transpose(one_hot_t, (2, 0, 1)), outputs_length)
